# Initial kernel scaffold; baseline (speedup 1.0000x reference)
import jax
import jax.numpy as jnp
from jax.experimental import pallas as pl


def kernel(x, edge_index, W1_rel, b1_rel, W1_root, W2_rel, b2_rel, W2_root):
    raise NotImplementedError("write your pallas kernel here")


# speedup vs baseline: 10.0894x; 10.0894x over previous
import functools

import jax
import jax.numpy as jnp
from jax import lax
from jax.experimental import pallas as pl
from jax.experimental.pallas import tpu as pltpu
from jax.experimental.pallas import tpu_sc as plsc

N = 10000
E = 320000
D = 128

NC = 2
NS = 16
NW = NC * NS

SB = 128
NB_PER_W = 80
E_PAD = NW * SB * NB_PER_W
EPW_ROWS = NB_PER_W

STRIPE = 632
N_ACC = NS * STRIPE

_mesh = plsc.VectorSubcoreMesh(core_axis_name="c", subcore_axis_name="s")


@functools.partial(
    pl.kernel,
    out_type=jax.ShapeDtypeStruct((NC, N_ACC, D), jnp.float32),
    mesh=_mesh,
    scratch_types=[
        pltpu.VMEM((EPW_ROWS, SB), jnp.int32),
        pltpu.VMEM((EPW_ROWS, SB), jnp.int32),
        pltpu.VMEM((SB, D), jnp.float32),
        pltpu.VMEM_SHARED((N_ACC, D), jnp.float32),
        pltpu.SemaphoreType.DMA,
    ],
)
def _sc_row_scatter(x_hbm, src_hbm, dst_hbm, zeros_hbm, out_hbm,
                    src_v, dst_v, rows_v, acc_sh, sem):
    cid = lax.axis_index("c")
    sid = lax.axis_index("s")
    wid = sid * NC + cid

    pltpu.sync_copy(zeros_hbm, acc_sh.at[pl.ds(sid * STRIPE, STRIPE)])

    pltpu.sync_copy(src_hbm.at[pl.ds(wid * EPW_ROWS, EPW_ROWS)], src_v)
    pltpu.sync_copy(dst_hbm.at[pl.ds(wid * EPW_ROWS, EPW_ROWS)], dst_v)

    plsc.subcore_barrier()

    def body(j):
        pltpu.async_copy(x_hbm.at[src_v.at[j]], rows_v, sem).wait()
        pltpu.sync_copy(rows_v, acc_sh.at[dst_v.at[j]], add=True)

    pl.loop(0, NB_PER_W)(body)

    plsc.subcore_barrier()

    pltpu.sync_copy(
        acc_sh.at[pl.ds(sid * STRIPE, STRIPE)],
        out_hbm.at[cid, pl.ds(sid * STRIPE, STRIPE)],
    )


@functools.partial(
    pl.kernel,
    out_type=jax.ShapeDtypeStruct((NC * N_ACC,), jnp.float32),
    mesh=_mesh,
    scratch_types=[
        pltpu.VMEM((EPW_ROWS, SB), jnp.int32),
        pltpu.VMEM((EPW_ROWS, SB), jnp.int32),
        pltpu.VMEM((SB,), jnp.float32),
        pltpu.VMEM((STRIPE,), jnp.float32),
        pltpu.VMEM_SHARED((N_ACC,), jnp.float32),
        pltpu.SemaphoreType.DMA,
    ],
)
def _sc_scalar_scatter(s_hbm, src_hbm, dst_hbm, zeros_hbm, out_hbm,
                       src_v, dst_v, vals_v, bounce_v, acc_sh, sem):
    cid = lax.axis_index("c")
    sid = lax.axis_index("s")
    wid = sid * NC + cid

    pltpu.sync_copy(zeros_hbm, bounce_v)
    pltpu.sync_copy(bounce_v, acc_sh.at[pl.ds(sid * STRIPE, STRIPE)])
    pltpu.sync_copy(src_hbm.at[pl.ds(wid * EPW_ROWS, EPW_ROWS)], src_v)
    pltpu.sync_copy(dst_hbm.at[pl.ds(wid * EPW_ROWS, EPW_ROWS)], dst_v)

    plsc.subcore_barrier()

    def body(j):
        pltpu.async_copy(s_hbm.at[src_v.at[j]], vals_v, sem).wait()
        pltpu.sync_copy(vals_v, acc_sh.at[dst_v.at[j]], add=True)

    pl.loop(0, NB_PER_W)(body)

    plsc.subcore_barrier()

    pltpu.sync_copy(acc_sh.at[pl.ds(sid * STRIPE, STRIPE)], bounce_v)
    pltpu.sync_copy(
        bounce_v,
        out_hbm.at[pl.ds(cid * N_ACC + sid * STRIPE, STRIPE)],
    )


_BLK = 2000
_DN = (((1,), (1,)), ((), ()))


def _tc_dense_body(parts_ref, x_ref, w1rel_ref, w1root_ref, b1_ref,
                   w2rel_ref, w2root_ref, s_ref, r_ref):
    agg = parts_ref[0] + parts_ref[1]
    ar = lax.dot_general(agg, w1rel_ref[...], _DN,
                         preferred_element_type=jnp.float32)
    xr = lax.dot_general(x_ref[...], w1root_ref[...], _DN,
                         preferred_element_type=jnp.float32)
    h = jnp.maximum(ar + b1_ref[...] + xr, 0.0)
    s_ref[...] = lax.dot_general(h, w2rel_ref[...], _DN,
                                 preferred_element_type=jnp.float32)
    r_ref[...] = lax.dot_general(h, w2root_ref[...], _DN,
                                 preferred_element_type=jnp.float32)


_tc_dense = pl.pallas_call(
    _tc_dense_body,
    grid=(N // _BLK,),
    in_specs=[
        pl.BlockSpec((NC, _BLK, D), lambda i: (0, i, 0)),
        pl.BlockSpec((_BLK, D), lambda i: (i, 0)),
        pl.BlockSpec((D, D), lambda i: (0, 0)),
        pl.BlockSpec((D, D), lambda i: (0, 0)),
        pl.BlockSpec((1, D), lambda i: (0, 0)),
        pl.BlockSpec((1, D), lambda i: (0, 0)),
        pl.BlockSpec((1, D), lambda i: (0, 0)),
    ],
    out_specs=[
        pl.BlockSpec((_BLK, 1), lambda i: (i, 0)),
        pl.BlockSpec((_BLK, 1), lambda i: (i, 0)),
    ],
    out_shape=[
        jax.ShapeDtypeStruct((N, 1), jnp.float32),
        jax.ShapeDtypeStruct((N, 1), jnp.float32),
    ],
)


def _tc_final_body(agg2_ref, r_ref, b2_ref, o_ref):
    z = agg2_ref[0:1, :] + agg2_ref[1:2, :] + r_ref[...] + b2_ref[...]
    o_ref[...] = 1.0 / (1.0 + jnp.exp(-z))


_tc_final = pl.pallas_call(
    _tc_final_body,
    out_shape=jax.ShapeDtypeStruct((1, N), jnp.float32),
)


def kernel(x, edge_index, W1_rel, b1_rel, W1_root, W2_rel, b2_rel, W2_root):
    src = edge_index[0]
    dst = edge_index[1]

    pad = E_PAD - E
    k = jnp.arange(pad, dtype=jnp.int32)
    src_p = jnp.concatenate([src, (k * 613) % N]).reshape(E_PAD // SB, SB)
    dst_p = jnp.concatenate([dst, N + (k % 8)]).reshape(E_PAD // SB, SB)

    zeros_rows = jnp.zeros((STRIPE, D), jnp.float32)
    parts = _sc_row_scatter(x, src_p, dst_p, zeros_rows)

    s, r = _tc_dense(parts, x,
                     W1_rel, W1_root,
                     b1_rel.reshape(1, D),
                     W2_rel, W2_root)

    zeros_1d = jnp.zeros((STRIPE,), jnp.float32)
    agg2 = _sc_scalar_scatter(s.reshape(N), src_p, dst_p, zeros_1d)
    agg2 = agg2.reshape(NC, N_ACC)

    out = _tc_final(agg2[:, :N], r.reshape(1, N), b2_rel.reshape(1, 1))
    return out.reshape(N, 1)
